# baseline (speedup 1.0000x reference)
import functools

import jax
import jax.numpy as jnp
from jax.experimental import pallas as pl
from jax.experimental.pallas import tpu as pltpu
from jax.experimental.pallas import tpu_sc as plsc

B = 4
N = 20000
C = 21
IMG_H = 512.0
IMG_W = 512.0
SCORE_THR = 0.3
IOU_THR = 0.5
MAX_DET = 100
NEG = -1e9

NPAD = 20480
CPAD = 24
KPAD = 128
R = B * CPAD

BUF = 1008
BUFP = 1024

import math as _math
LADDER = tuple(
    [0.0] + [float(1.0 / (1.0 + _math.exp(-0.5 * k))) for k in range(1, 16)])


def _decode_rows(rel_ref, anch_ref):
    ya0 = anch_ref[0:1, :]
    xa0 = anch_ref[1:2, :]
    ya1 = anch_ref[2:3, :]
    xa1 = anch_ref[3:4, :]
    ycenter_a = (ya0 + ya1) / 2.0
    xcenter_a = (xa0 + xa1) / 2.0
    ha = ya1 - ya0
    wa = xa1 - xa0
    ty = rel_ref[0, 0:1, :] / 10.0
    tx = rel_ref[0, 1:2, :] / 10.0
    th = rel_ref[0, 2:3, :] / 5.0
    tw = rel_ref[0, 3:4, :] / 5.0
    h = jnp.exp(th) * ha
    w = jnp.exp(tw) * wa
    yc = ty * ha + ycenter_a
    xc = tx * wa + xcenter_a
    ymin = jnp.clip(yc - h / 2.0, 0.0, IMG_H)
    xmin = jnp.clip(xc - w / 2.0, 0.0, IMG_W)
    ymax = jnp.clip(yc + h / 2.0, 0.0, IMG_H)
    xmax = jnp.clip(xc + w / 2.0, 0.0, IMG_W)
    return ymin, xmin, ymax, xmax


def _prep_body(rel_ref, anch_ref, sc_ref, prob_ref, coord_ref, cnt_ref):
    ymin, xmin, ymax, xmax = _decode_rows(rel_ref, anch_ref)
    zrow = jnp.zeros((1, NPAD), jnp.float32)
    coord_ref[0] = jnp.concatenate(
        [ymin, xmin, ymax, xmax, zrow, zrow, zrow, zrow], axis=0)
    prob = jax.nn.sigmoid(sc_ref[0])
    p = jnp.where(prob > SCORE_THR, prob, NEG)
    prob_ref[0] = p
    cnts = [jnp.sum((p > q).astype(jnp.float32), axis=1, keepdims=True)
            for q in LADDER]
    cnt_ref[0] = jnp.concatenate(cnts, axis=1)


def _sc_body(prob_hbm, coord_hbm, cnt_hbm, qv_hbm,
             cprob_hbm, cy0_hbm, cx0_hbm, cy1_hbm, cx1_hbm,
             probs_t, y0_t, x0_t, y1_t, x1_t,
             cprob_t, cglob_t, cy0_t, cx0_t, cy1_t, cx1_t, cnt_t, qv_t):
    nc = 2
    wid = jax.lax.axis_index("s") * nc + jax.lax.axis_index("c")
    b = wid // (CPAD // 3)
    iota16 = jax.lax.iota(jnp.int32, 16)
    pltpu.sync_copy(qv_hbm, qv_t)
    qvals = qv_t[...]

    pltpu.sync_copy(coord_hbm.at[b, 0], y0_t)
    pltpu.sync_copy(coord_hbm.at[b, 1], x0_t)
    pltpu.sync_copy(coord_hbm.at[b, 2], y1_t)
    pltpu.sync_copy(coord_hbm.at[b, 3], x1_t)

    def do_row(i, _):
        r = wid * 3 + i
        pltpu.sync_copy(prob_hbm.at[r], probs_t)
        pltpu.sync_copy(cnt_hbm.at[r], cnt_t)

        @plsc.parallel_loop(0, BUFP // 16, unroll=8)
        def _(g):
            sl = pl.ds(g * 16, 16)
            cprob_t[sl] = jnp.full((16,), NEG, jnp.float32)
            cglob_t[sl] = jnp.zeros((16,), jnp.int32)

        counts = cnt_t[...]
        cutv = jnp.min(jnp.where(counts <= float(BUF), qvals, 2.0))

        def comp(g, pos):
            p16 = probs_t[pl.ds(g * 16, 16)]
            mask = p16 > cutv
            tgt = pos + plsc.cumsum(mask.astype(jnp.int32)) - 1
            plsc.store_scatter(cprob_t, [tgt], p16, mask=mask)
            plsc.store_scatter(cglob_t, [tgt], g * 16 + iota16, mask=mask)
            return pos + plsc.all_reduce_population_count(mask)
        jax.lax.fori_loop(0, NPAD // 16, comp,
                          jnp.zeros((16,), jnp.int32), unroll=16)

        @plsc.parallel_loop(0, BUFP // 16, unroll=8)
        def _(g):
            sl = pl.ds(g * 16, 16)
            gi = cglob_t[sl]
            cy0_t[sl] = plsc.load_gather(y0_t, [gi])
            cx0_t[sl] = plsc.load_gather(x0_t, [gi])
            cy1_t[sl] = plsc.load_gather(y1_t, [gi])
            cx1_t[sl] = plsc.load_gather(x1_t, [gi])

        pltpu.sync_copy(cprob_t, cprob_hbm.at[r])
        pltpu.sync_copy(cy0_t, cy0_hbm.at[r])
        pltpu.sync_copy(cx0_t, cx0_hbm.at[r])
        pltpu.sync_copy(cy1_t, cy1_hbm.at[r])
        pltpu.sync_copy(cx1_t, cx1_hbm.at[r])
        return 0

    jax.lax.fori_loop(0, 3, do_row, 0)


def _make_sc_compact():
    mesh = plsc.VectorSubcoreMesh(core_axis_name="c", subcore_axis_name="s")
    f32, i32 = jnp.float32, jnp.int32
    return pl.kernel(
        _sc_body,
        out_type=[
            jax.ShapeDtypeStruct((R, BUFP), f32),
            jax.ShapeDtypeStruct((R, BUFP), f32),
            jax.ShapeDtypeStruct((R, BUFP), f32),
            jax.ShapeDtypeStruct((R, BUFP), f32),
            jax.ShapeDtypeStruct((R, BUFP), f32),
        ],
        mesh=mesh,
        compiler_params=pltpu.CompilerParams(needs_layout_passes=False),
        scratch_types=[
            pltpu.VMEM((NPAD,), f32),
            pltpu.VMEM((NPAD,), f32),
            pltpu.VMEM((NPAD,), f32),
            pltpu.VMEM((NPAD,), f32),
            pltpu.VMEM((NPAD,), f32),
            pltpu.VMEM((BUFP,), f32),
            pltpu.VMEM((BUFP,), i32),
            pltpu.VMEM((BUFP,), f32),
            pltpu.VMEM((BUFP,), f32),
            pltpu.VMEM((BUFP,), f32),
            pltpu.VMEM((BUFP,), f32),
            pltpu.VMEM((16,), f32),
            pltpu.VMEM((16,), f32),
        ],
    )


def _merge_loop(ssc, sy0, sx0, sy1, sx1, iota_k):
    row_iota = jax.lax.broadcasted_iota(jnp.int32, (CPAD, KPAD), 0).astype(jnp.float32)

    def merge(j, carry):
        ssc, oy0, ox0, oy1, ox1, osc, olb = carry
        m2 = jnp.max(ssc)
        flat = row_iota * float(KPAD) + iota_k
        fidx = jnp.min(jnp.where(ssc == m2, flat, float(CPAD * KPAD)))
        oneh2 = flat == fidx
        valid = m2 > NEG / 2.0
        gy0 = jnp.sum(jnp.where(oneh2, sy0, 0.0))
        gx0 = jnp.sum(jnp.where(oneh2, sx0, 0.0))
        gy1 = jnp.sum(jnp.where(oneh2, sy1, 0.0))
        gx1 = jnp.sum(jnp.where(oneh2, sx1, 0.0))
        glb = jnp.sum(jnp.where(oneh2, row_iota, 0.0))
        colm = iota_k == j
        oy0 = jnp.where(colm, jnp.where(valid, gy0, 0.0), oy0)
        ox0 = jnp.where(colm, jnp.where(valid, gx0, 0.0), ox0)
        oy1 = jnp.where(colm, jnp.where(valid, gy1, 0.0), oy1)
        ox1 = jnp.where(colm, jnp.where(valid, gx1, 0.0), ox1)
        osc = jnp.where(colm, jnp.where(valid, m2, 0.0), osc)
        olb = jnp.where(colm, jnp.where(valid, glb, 0.0), olb)
        ssc = jnp.where(oneh2, NEG, ssc)
        return ssc, oy0, ox0, oy1, ox1, osc, olb

    zrow = jnp.zeros((1, KPAD), jnp.float32)
    return jax.lax.fori_loop(
        0, MAX_DET, merge, (ssc, zrow, zrow, zrow, zrow, zrow, zrow))


def _nms_core(sc, ymin, xmin, ymax, xmax, area, iota_n, iota_k):
    rows = sc.shape[0]
    neg_k = jnp.full((rows, KPAD), NEG, jnp.float32)
    zero_k = jnp.zeros((rows, KPAD), jnp.float32)
    width = sc.shape[-1]

    def pick(k, carry):
        sc, ssc, sy0, sx0, sy1, sx1 = carry
        m = jnp.max(sc, axis=1, keepdims=True)
        is_max = sc == m
        idx = jnp.min(jnp.where(is_max, iota_n, float(width)),
                      axis=1, keepdims=True)
        oneh = iota_n == idx
        by0 = jnp.sum(jnp.where(oneh, ymin, 0.0), axis=1, keepdims=True)
        bx0 = jnp.sum(jnp.where(oneh, xmin, 0.0), axis=1, keepdims=True)
        by1 = jnp.sum(jnp.where(oneh, ymax, 0.0), axis=1, keepdims=True)
        bx1 = jnp.sum(jnp.where(oneh, xmax, 0.0), axis=1, keepdims=True)
        yy1 = jnp.maximum(by0, ymin)
        xx1 = jnp.maximum(bx0, xmin)
        yy2 = jnp.minimum(by1, ymax)
        xx2 = jnp.minimum(bx1, xmax)
        inter = jnp.maximum(yy2 - yy1, 0.0) * jnp.maximum(xx2 - xx1, 0.0)
        a1 = (by1 - by0) * (bx1 - bx0)
        iou = inter / (a1 + area - inter + 1e-8)
        sc = jnp.where((iou > IOU_THR) | oneh, NEG, sc)
        colm = iota_k == k
        ssc = jnp.where(colm, m, ssc)
        sy0 = jnp.where(colm, by0, sy0)
        sx0 = jnp.where(colm, bx0, sx0)
        sy1 = jnp.where(colm, by1, sy1)
        sx1 = jnp.where(colm, bx1, sx1)
        return sc, ssc, sy0, sx0, sy1, sx1

    return jax.lax.fori_loop(
        0, MAX_DET, pick, (sc, neg_k, zero_k, zero_k, zero_k, zero_k))


def _rnms_body(cprob_ref, cy0_ref, cx0_ref, cy1_ref, cx1_ref, cnt_ref,
               out_ref):
    sc = cprob_ref[...]
    y0 = cy0_ref[...]
    x0 = cx0_ref[...]
    y1 = cy1_ref[...]
    x1 = cx1_ref[...]
    area = (y1 - y0) * (x1 - x0)
    iota_n = jax.lax.broadcasted_iota(jnp.int32, (1, BUFP), 1).astype(jnp.float32)
    iota_k = jax.lax.broadcasted_iota(jnp.int32, (1, KPAD), 1).astype(jnp.float32)

    _, ssc, sy0, sx0, sy1, sx1 = _nms_core(sc, y0, x0, y1, x1, area,
                                           iota_n, iota_k)

    ssc3 = ssc.reshape(B, CPAD, KPAD)
    sy0_3 = sy0.reshape(B, CPAD, KPAD)
    sx0_3 = sx0.reshape(B, CPAD, KPAD)
    sy1_3 = sy1.reshape(B, CPAD, KPAD)
    sx1_3 = sx1.reshape(B, CPAD, KPAD)

    picks = jnp.sum((ssc3 > NEG / 2.0).astype(jnp.float32),
                    axis=2, keepdims=True)
    total = cnt_ref[:, :, 0:1]
    okrow = (picks >= float(MAX_DET) - 0.5) | (total <= float(BUF) - 0.5)
    flag3 = jnp.min(jnp.where(okrow, 1.0, 0.0), axis=1,
                    keepdims=True)
    nv3 = jnp.minimum(jnp.sum(picks, axis=1, keepdims=True),
                      float(MAX_DET))

    iota_c3 = jax.lax.broadcasted_iota(
        jnp.int32, (1, CPAD, KPAD), 1).astype(jnp.float32)
    iota_k3 = jax.lax.broadcasted_iota(
        jnp.int32, (1, 1, KPAD), 2).astype(jnp.float32)
    flat3 = iota_c3 * float(KPAD) + iota_k3

    def merge(j, carry):
        ssc3, oy0, ox0, oy1, ox1, osc, olb = carry
        m2 = jnp.max(jnp.max(ssc3, axis=2), axis=1).reshape(B, 1, 1)
        cand = jnp.where(ssc3 == m2, flat3, float(CPAD * KPAD))
        fidx = jnp.min(jnp.min(cand, axis=2), axis=1).reshape(B, 1, 1)
        oneh2 = flat3 == fidx
        valid = m2 > NEG / 2.0

        def pick3(v):
            return jnp.sum(jnp.sum(jnp.where(oneh2, v, 0.0), axis=2),
                           axis=1).reshape(B, 1, 1)
        gy0 = pick3(sy0_3)
        gx0 = pick3(sx0_3)
        gy1 = pick3(sy1_3)
        gx1 = pick3(sx1_3)
        glb = pick3(jnp.broadcast_to(iota_c3, (B, CPAD, KPAD)))
        colm = iota_k3 == j
        oy0 = jnp.where(colm, jnp.where(valid, gy0, 0.0), oy0)
        ox0 = jnp.where(colm, jnp.where(valid, gx0, 0.0), ox0)
        oy1 = jnp.where(colm, jnp.where(valid, gy1, 0.0), oy1)
        ox1 = jnp.where(colm, jnp.where(valid, gx1, 0.0), ox1)
        osc = jnp.where(colm, jnp.where(valid, m2, 0.0), osc)
        olb = jnp.where(colm, jnp.where(valid, glb, 0.0), olb)
        ssc3 = jnp.where(oneh2, NEG, ssc3)
        return ssc3, oy0, ox0, oy1, ox1, osc, olb

    zrow = jnp.zeros((B, 1, KPAD), jnp.float32)
    _, oy0, ox0, oy1, ox1, osc, olb = jax.lax.fori_loop(
        0, MAX_DET, merge, (ssc3, zrow, zrow, zrow, zrow, zrow, zrow))

    out_ref[...] = jnp.concatenate(
        [oy0, ox0, oy1, ox1, osc, olb,
         jnp.broadcast_to(nv3, (B, 1, KPAD)),
         jnp.broadcast_to(flag3, (B, 1, KPAD))], axis=1)


def _full_body(rel_ref, anch_ref, sc_ref, out_ref):
    ymin, xmin, ymax, xmax = _decode_rows(rel_ref, anch_ref)
    area = (ymax - ymin) * (xmax - xmin)
    prob = jax.nn.sigmoid(sc_ref[0])
    sc = jnp.where(prob > SCORE_THR, prob, NEG)
    iota_n = jax.lax.broadcasted_iota(jnp.int32, (1, NPAD), 1).astype(jnp.float32)
    iota_k = jax.lax.broadcasted_iota(jnp.int32, (1, KPAD), 1).astype(jnp.float32)
    _, ssc, sy0, sx0, sy1, sx1 = _nms_core(sc, ymin, xmin, ymax, xmax, area,
                                           iota_n, iota_k)
    nvalid = jnp.minimum(jnp.sum((ssc > NEG / 2.0).astype(jnp.float32)),
                         float(MAX_DET))
    _, oy0, ox0, oy1, ox1, osc, olb = _merge_loop(ssc, sy0, sx0, sy1, sx1,
                                                  iota_k)
    zrow = jnp.zeros((1, KPAD), jnp.float32)
    out_ref[0] = jnp.concatenate(
        [oy0, ox0, oy1, ox1, osc, olb,
         jnp.full((1, KPAD), nvalid, jnp.float32), zrow], axis=0)


def _postprocess(o):
    out_boxes = jnp.stack([o[:, 0, :MAX_DET], o[:, 1, :MAX_DET],
                           o[:, 2, :MAX_DET], o[:, 3, :MAX_DET]], axis=-1)
    out_scores = o[:, 4, :MAX_DET]
    out_labels = o[:, 5, :MAX_DET]
    num_valid = o[:, 6, 0].astype(jnp.int32)
    return out_boxes, out_scores, out_labels, num_valid


def _full_path(relT, anchT, scT):
    o = pl.pallas_call(
        _full_body,
        grid=(B,),
        in_specs=[
            pl.BlockSpec((1, 8, NPAD), lambda b: (b, 0, 0)),
            pl.BlockSpec((8, NPAD), lambda b: (0, 0)),
            pl.BlockSpec((1, CPAD, NPAD), lambda b: (b, 0, 0)),
        ],
        out_specs=pl.BlockSpec((1, 8, KPAD), lambda b: (b, 0, 0)),
        out_shape=jax.ShapeDtypeStruct((B, 8, KPAD), jnp.float32),
        compiler_params=pltpu.CompilerParams(
            dimension_semantics=("arbitrary",)),
    )(relT, anchT, scT)
    return _postprocess(o)


def kernel(rel_codes, scores, anchors):
    relT = jnp.pad(jnp.transpose(rel_codes, (0, 2, 1)),
                   ((0, 0), (0, 4), (0, NPAD - N)))
    anchT = jnp.pad(jnp.transpose(anchors, (1, 0)),
                    ((0, 4), (0, NPAD - N)))
    scT = jnp.pad(jnp.transpose(scores, (0, 2, 1)),
                  ((0, 0), (0, CPAD - C), (0, NPAD - N)),
                  constant_values=NEG)

    probs, coords, cnts = pl.pallas_call(
        _prep_body,
        grid=(B,),
        in_specs=[
            pl.BlockSpec((1, 8, NPAD), lambda b: (b, 0, 0)),
            pl.BlockSpec((8, NPAD), lambda b: (0, 0)),
            pl.BlockSpec((1, CPAD, NPAD), lambda b: (b, 0, 0)),
        ],
        out_specs=[
            pl.BlockSpec((1, CPAD, NPAD), lambda b: (b, 0, 0)),
            pl.BlockSpec((1, 8, NPAD), lambda b: (b, 0, 0)),
            pl.BlockSpec((1, CPAD, 16), lambda b: (b, 0, 0)),
        ],
        out_shape=[
            jax.ShapeDtypeStruct((B, CPAD, NPAD), jnp.float32),
            jax.ShapeDtypeStruct((B, 8, NPAD), jnp.float32),
            jax.ShapeDtypeStruct((B, CPAD, 16), jnp.float32),
        ],
        compiler_params=pltpu.CompilerParams(
            dimension_semantics=("arbitrary",)),
    )(relT, anchT, scT)

    sc_compact = _make_sc_compact()
    cprob, cy0, cx0, cy1, cx1 = sc_compact(
        probs.reshape(R, NPAD), coords, cnts.reshape(R, 16),
        jnp.asarray(LADDER, jnp.float32))

    o = pl.pallas_call(
        _rnms_body,
        out_shape=jax.ShapeDtypeStruct((B, 8, KPAD), jnp.float32),
    )(cprob, cy0, cx0, cy1, cx1, cnts)

    ok = jnp.min(o[:, 7, 0]) > 0.5
    fast = _postprocess(o)
    return jax.lax.cond(ok, lambda: fast,
                        lambda: _full_path(relT, anchT, scT))


# speedup vs baseline: 1.0102x; 1.0102x over previous
import functools

import jax
import jax.numpy as jnp
from jax.experimental import pallas as pl
from jax.experimental.pallas import tpu as pltpu
from jax.experimental.pallas import tpu_sc as plsc

B = 4
N = 20000
C = 21
IMG_H = 512.0
IMG_W = 512.0
SCORE_THR = 0.3
IOU_THR = 0.5
MAX_DET = 100
NEG = -1e9

NPAD = 20480
CPAD = 24
KPAD = 128
R = B * CPAD

BUF = 1008
BUFP = 1024

import math as _math
LADDER = tuple(
    [0.0] + [float(1.0 / (1.0 + _math.exp(-0.5 * k))) for k in range(1, 16)])


def _decode_rows(rel_ref, anch_ref):
    ya0 = anch_ref[0:1, :]
    xa0 = anch_ref[1:2, :]
    ya1 = anch_ref[2:3, :]
    xa1 = anch_ref[3:4, :]
    ycenter_a = (ya0 + ya1) / 2.0
    xcenter_a = (xa0 + xa1) / 2.0
    ha = ya1 - ya0
    wa = xa1 - xa0
    ty = rel_ref[0, 0:1, :] / 10.0
    tx = rel_ref[0, 1:2, :] / 10.0
    th = rel_ref[0, 2:3, :] / 5.0
    tw = rel_ref[0, 3:4, :] / 5.0
    h = jnp.exp(th) * ha
    w = jnp.exp(tw) * wa
    yc = ty * ha + ycenter_a
    xc = tx * wa + xcenter_a
    ymin = jnp.clip(yc - h / 2.0, 0.0, IMG_H)
    xmin = jnp.clip(xc - w / 2.0, 0.0, IMG_W)
    ymax = jnp.clip(yc + h / 2.0, 0.0, IMG_H)
    xmax = jnp.clip(xc + w / 2.0, 0.0, IMG_W)
    return ymin, xmin, ymax, xmax


def _prep_body(rel_ref, anch_ref, sc_ref, prob_ref, coord_ref, cnt_ref):
    ymin, xmin, ymax, xmax = _decode_rows(rel_ref, anch_ref)
    zrow = jnp.zeros((1, NPAD), jnp.float32)
    coord_ref[0] = jnp.concatenate(
        [ymin, xmin, ymax, xmax, zrow, zrow, zrow, zrow], axis=0)
    prob = jax.nn.sigmoid(sc_ref[0])
    p = jnp.where(prob > SCORE_THR, prob, NEG)
    prob_ref[0] = p
    cnts = [jnp.sum((p > q).astype(jnp.float32), axis=1, keepdims=True)
            for q in LADDER]
    cnt_ref[0] = jnp.concatenate(cnts, axis=1)


def _sc_body(prob_hbm, coord_hbm, cnt_hbm, qv_hbm,
             cprob_hbm, cy0_hbm, cx0_hbm, cy1_hbm, cx1_hbm,
             probs_a, probs_b, y0_t, x0_t, y1_t, x1_t,
             cprob_t, cglob_t, cy0_t, cx0_t, cy1_t, cx1_t, cnt3_t, qv_t,
             sem_in, sem_out):
    nc = 2
    wid = jax.lax.axis_index("s") * nc + jax.lax.axis_index("c")
    b = wid // (CPAD // 3)
    iota16 = jax.lax.iota(jnp.int32, 16)
    pltpu.sync_copy(qv_hbm, qv_t)
    qvals = qv_t[...]

    pltpu.sync_copy(coord_hbm.at[b, 0], y0_t)
    pltpu.sync_copy(coord_hbm.at[b, 1], x0_t)
    pltpu.sync_copy(coord_hbm.at[b, 2], y1_t)
    pltpu.sync_copy(coord_hbm.at[b, 3], x1_t)

    in_copy = pltpu.async_copy(prob_hbm.at[wid * 3], probs_a, sem_in)
    out_descs = []
    for i in range(3):
        cur = probs_a if i % 2 == 0 else probs_b
        in_copy.wait()
        if i < 2:
            nxt = probs_b if i % 2 == 0 else probs_a
            in_copy = pltpu.async_copy(prob_hbm.at[wid * 3 + i + 1],
                                       nxt, sem_in)
        for d in out_descs:
            d.wait()

        @plsc.parallel_loop(0, BUFP // 16, unroll=8)
        def _(g):
            sl = pl.ds(g * 16, 16)
            cprob_t[sl] = jnp.full((16,), NEG, jnp.float32)
            cglob_t[sl] = jnp.zeros((16,), jnp.int32)

        pltpu.sync_copy(cnt_hbm.at[wid * 3 + i], cnt3_t)
        counts = cnt3_t[...]
        cutv = jnp.min(jnp.where(counts <= float(BUF), qvals, 2.0))

        def comp(g, pos):
            p16 = cur[pl.ds(g * 16, 16)]
            mask = p16 > cutv
            tgt = pos + plsc.cumsum(mask.astype(jnp.int32)) - 1
            plsc.store_scatter(cprob_t, [tgt], p16, mask=mask)
            plsc.store_scatter(cglob_t, [tgt], g * 16 + iota16, mask=mask)
            return pos + plsc.all_reduce_population_count(mask)
        jax.lax.fori_loop(0, NPAD // 16, comp,
                          jnp.zeros((16,), jnp.int32), unroll=16)

        @plsc.parallel_loop(0, BUFP // 16, unroll=8)
        def _(g):
            sl = pl.ds(g * 16, 16)
            gi = cglob_t[sl]
            cy0_t[sl] = plsc.load_gather(y0_t, [gi])
            cx0_t[sl] = plsc.load_gather(x0_t, [gi])
            cy1_t[sl] = plsc.load_gather(y1_t, [gi])
            cx1_t[sl] = plsc.load_gather(x1_t, [gi])

        r = wid * 3 + i
        out_descs = [
            pltpu.async_copy(cprob_t, cprob_hbm.at[r], sem_out),
            pltpu.async_copy(cy0_t, cy0_hbm.at[r], sem_out),
            pltpu.async_copy(cx0_t, cx0_hbm.at[r], sem_out),
            pltpu.async_copy(cy1_t, cy1_hbm.at[r], sem_out),
            pltpu.async_copy(cx1_t, cx1_hbm.at[r], sem_out),
        ]
    for d in out_descs:
        d.wait()


def _make_sc_compact():
    mesh = plsc.VectorSubcoreMesh(core_axis_name="c", subcore_axis_name="s")
    f32, i32 = jnp.float32, jnp.int32
    return pl.kernel(
        _sc_body,
        out_type=[
            jax.ShapeDtypeStruct((R, BUFP), f32),
            jax.ShapeDtypeStruct((R, BUFP), f32),
            jax.ShapeDtypeStruct((R, BUFP), f32),
            jax.ShapeDtypeStruct((R, BUFP), f32),
            jax.ShapeDtypeStruct((R, BUFP), f32),
        ],
        mesh=mesh,
        compiler_params=pltpu.CompilerParams(needs_layout_passes=False),
        scratch_types=[
            pltpu.VMEM((NPAD,), f32),
            pltpu.VMEM((NPAD,), f32),
            pltpu.VMEM((NPAD,), f32),
            pltpu.VMEM((NPAD,), f32),
            pltpu.VMEM((NPAD,), f32),
            pltpu.VMEM((NPAD,), f32),
            pltpu.VMEM((BUFP,), f32),
            pltpu.VMEM((BUFP,), i32),
            pltpu.VMEM((BUFP,), f32),
            pltpu.VMEM((BUFP,), f32),
            pltpu.VMEM((BUFP,), f32),
            pltpu.VMEM((BUFP,), f32),
            pltpu.VMEM((16,), f32),
            pltpu.VMEM((16,), f32),
            pltpu.SemaphoreType.DMA,
            pltpu.SemaphoreType.DMA,
        ],
    )


def _merge_loop(ssc, sy0, sx0, sy1, sx1, iota_k):
    row_iota = jax.lax.broadcasted_iota(jnp.int32, (CPAD, KPAD), 0).astype(jnp.float32)

    def merge(j, carry):
        ssc, oy0, ox0, oy1, ox1, osc, olb = carry
        m2 = jnp.max(ssc)
        flat = row_iota * float(KPAD) + iota_k
        fidx = jnp.min(jnp.where(ssc == m2, flat, float(CPAD * KPAD)))
        oneh2 = flat == fidx
        valid = m2 > NEG / 2.0
        gy0 = jnp.sum(jnp.where(oneh2, sy0, 0.0))
        gx0 = jnp.sum(jnp.where(oneh2, sx0, 0.0))
        gy1 = jnp.sum(jnp.where(oneh2, sy1, 0.0))
        gx1 = jnp.sum(jnp.where(oneh2, sx1, 0.0))
        glb = jnp.sum(jnp.where(oneh2, row_iota, 0.0))
        colm = iota_k == j
        oy0 = jnp.where(colm, jnp.where(valid, gy0, 0.0), oy0)
        ox0 = jnp.where(colm, jnp.where(valid, gx0, 0.0), ox0)
        oy1 = jnp.where(colm, jnp.where(valid, gy1, 0.0), oy1)
        ox1 = jnp.where(colm, jnp.where(valid, gx1, 0.0), ox1)
        osc = jnp.where(colm, jnp.where(valid, m2, 0.0), osc)
        olb = jnp.where(colm, jnp.where(valid, glb, 0.0), olb)
        ssc = jnp.where(oneh2, NEG, ssc)
        return ssc, oy0, ox0, oy1, ox1, osc, olb

    zrow = jnp.zeros((1, KPAD), jnp.float32)
    return jax.lax.fori_loop(
        0, MAX_DET, merge, (ssc, zrow, zrow, zrow, zrow, zrow, zrow))


def _nms_core(sc, ymin, xmin, ymax, xmax, area, iota_n, iota_k):
    rows = sc.shape[0]
    neg_k = jnp.full((rows, KPAD), NEG, jnp.float32)
    zero_k = jnp.zeros((rows, KPAD), jnp.float32)
    width = sc.shape[-1]

    def pick(k, carry):
        sc, ssc, sy0, sx0, sy1, sx1 = carry
        m = jnp.max(sc, axis=1, keepdims=True)
        is_max = sc == m
        idx = jnp.min(jnp.where(is_max, iota_n, float(width)),
                      axis=1, keepdims=True)
        oneh = iota_n == idx
        by0 = jnp.sum(jnp.where(oneh, ymin, 0.0), axis=1, keepdims=True)
        bx0 = jnp.sum(jnp.where(oneh, xmin, 0.0), axis=1, keepdims=True)
        by1 = jnp.sum(jnp.where(oneh, ymax, 0.0), axis=1, keepdims=True)
        bx1 = jnp.sum(jnp.where(oneh, xmax, 0.0), axis=1, keepdims=True)
        yy1 = jnp.maximum(by0, ymin)
        xx1 = jnp.maximum(bx0, xmin)
        yy2 = jnp.minimum(by1, ymax)
        xx2 = jnp.minimum(bx1, xmax)
        inter = jnp.maximum(yy2 - yy1, 0.0) * jnp.maximum(xx2 - xx1, 0.0)
        a1 = (by1 - by0) * (bx1 - bx0)
        iou = inter / (a1 + area - inter + 1e-8)
        sc = jnp.where((iou > IOU_THR) | oneh, NEG, sc)
        colm = iota_k == k
        ssc = jnp.where(colm, m, ssc)
        sy0 = jnp.where(colm, by0, sy0)
        sx0 = jnp.where(colm, bx0, sx0)
        sy1 = jnp.where(colm, by1, sy1)
        sx1 = jnp.where(colm, bx1, sx1)
        return sc, ssc, sy0, sx0, sy1, sx1

    return jax.lax.fori_loop(
        0, MAX_DET, pick, (sc, neg_k, zero_k, zero_k, zero_k, zero_k))


def _rnms_body(cprob_ref, cy0_ref, cx0_ref, cy1_ref, cx1_ref, cnt_ref,
               out_ref):
    sc = cprob_ref[...]
    y0 = cy0_ref[...]
    x0 = cx0_ref[...]
    y1 = cy1_ref[...]
    x1 = cx1_ref[...]
    area = (y1 - y0) * (x1 - x0)
    iota_n = jax.lax.broadcasted_iota(jnp.int32, (1, BUFP), 1).astype(jnp.float32)
    iota_k = jax.lax.broadcasted_iota(jnp.int32, (1, KPAD), 1).astype(jnp.float32)

    _, ssc, sy0, sx0, sy1, sx1 = _nms_core(sc, y0, x0, y1, x1, area,
                                           iota_n, iota_k)

    ssc3 = ssc.reshape(B, CPAD, KPAD)
    sy0_3 = sy0.reshape(B, CPAD, KPAD)
    sx0_3 = sx0.reshape(B, CPAD, KPAD)
    sy1_3 = sy1.reshape(B, CPAD, KPAD)
    sx1_3 = sx1.reshape(B, CPAD, KPAD)

    picks = jnp.sum((ssc3 > NEG / 2.0).astype(jnp.float32),
                    axis=2, keepdims=True)
    total = cnt_ref[:, :, 0:1]
    okrow = (picks >= float(MAX_DET) - 0.5) | (total <= float(BUF) - 0.5)
    flag3 = jnp.min(jnp.where(okrow, 1.0, 0.0), axis=1,
                    keepdims=True)
    nv3 = jnp.minimum(jnp.sum(picks, axis=1, keepdims=True),
                      float(MAX_DET))

    iota_c3 = jax.lax.broadcasted_iota(
        jnp.int32, (1, CPAD, KPAD), 1).astype(jnp.float32)
    iota_k3 = jax.lax.broadcasted_iota(
        jnp.int32, (1, 1, KPAD), 2).astype(jnp.float32)
    flat3 = iota_c3 * float(KPAD) + iota_k3

    def merge(j, carry):
        ssc3, oy0, ox0, oy1, ox1, osc, olb = carry
        m2 = jnp.max(jnp.max(ssc3, axis=2), axis=1).reshape(B, 1, 1)
        cand = jnp.where(ssc3 == m2, flat3, float(CPAD * KPAD))
        fidx = jnp.min(jnp.min(cand, axis=2), axis=1).reshape(B, 1, 1)
        oneh2 = flat3 == fidx
        valid = m2 > NEG / 2.0

        def pick3(v):
            return jnp.sum(jnp.sum(jnp.where(oneh2, v, 0.0), axis=2),
                           axis=1).reshape(B, 1, 1)
        gy0 = pick3(sy0_3)
        gx0 = pick3(sx0_3)
        gy1 = pick3(sy1_3)
        gx1 = pick3(sx1_3)
        glb = pick3(jnp.broadcast_to(iota_c3, (B, CPAD, KPAD)))
        colm = iota_k3 == j
        oy0 = jnp.where(colm, jnp.where(valid, gy0, 0.0), oy0)
        ox0 = jnp.where(colm, jnp.where(valid, gx0, 0.0), ox0)
        oy1 = jnp.where(colm, jnp.where(valid, gy1, 0.0), oy1)
        ox1 = jnp.where(colm, jnp.where(valid, gx1, 0.0), ox1)
        osc = jnp.where(colm, jnp.where(valid, m2, 0.0), osc)
        olb = jnp.where(colm, jnp.where(valid, glb, 0.0), olb)
        ssc3 = jnp.where(oneh2, NEG, ssc3)
        return ssc3, oy0, ox0, oy1, ox1, osc, olb

    zrow = jnp.zeros((B, 1, KPAD), jnp.float32)
    _, oy0, ox0, oy1, ox1, osc, olb = jax.lax.fori_loop(
        0, MAX_DET, merge, (ssc3, zrow, zrow, zrow, zrow, zrow, zrow))

    out_ref[...] = jnp.concatenate(
        [oy0, ox0, oy1, ox1, osc, olb,
         jnp.broadcast_to(nv3, (B, 1, KPAD)),
         jnp.broadcast_to(flag3, (B, 1, KPAD))], axis=1)


def _full_body(rel_ref, anch_ref, sc_ref, out_ref):
    ymin, xmin, ymax, xmax = _decode_rows(rel_ref, anch_ref)
    area = (ymax - ymin) * (xmax - xmin)
    prob = jax.nn.sigmoid(sc_ref[0])
    sc = jnp.where(prob > SCORE_THR, prob, NEG)
    iota_n = jax.lax.broadcasted_iota(jnp.int32, (1, NPAD), 1).astype(jnp.float32)
    iota_k = jax.lax.broadcasted_iota(jnp.int32, (1, KPAD), 1).astype(jnp.float32)
    _, ssc, sy0, sx0, sy1, sx1 = _nms_core(sc, ymin, xmin, ymax, xmax, area,
                                           iota_n, iota_k)
    nvalid = jnp.minimum(jnp.sum((ssc > NEG / 2.0).astype(jnp.float32)),
                         float(MAX_DET))
    _, oy0, ox0, oy1, ox1, osc, olb = _merge_loop(ssc, sy0, sx0, sy1, sx1,
                                                  iota_k)
    zrow = jnp.zeros((1, KPAD), jnp.float32)
    out_ref[0] = jnp.concatenate(
        [oy0, ox0, oy1, ox1, osc, olb,
         jnp.full((1, KPAD), nvalid, jnp.float32), zrow], axis=0)


def _postprocess(o):
    out_boxes = jnp.stack([o[:, 0, :MAX_DET], o[:, 1, :MAX_DET],
                           o[:, 2, :MAX_DET], o[:, 3, :MAX_DET]], axis=-1)
    out_scores = o[:, 4, :MAX_DET]
    out_labels = o[:, 5, :MAX_DET]
    num_valid = o[:, 6, 0].astype(jnp.int32)
    return out_boxes, out_scores, out_labels, num_valid


def _full_path(relT, anchT, scT):
    o = pl.pallas_call(
        _full_body,
        grid=(B,),
        in_specs=[
            pl.BlockSpec((1, 8, NPAD), lambda b: (b, 0, 0)),
            pl.BlockSpec((8, NPAD), lambda b: (0, 0)),
            pl.BlockSpec((1, CPAD, NPAD), lambda b: (b, 0, 0)),
        ],
        out_specs=pl.BlockSpec((1, 8, KPAD), lambda b: (b, 0, 0)),
        out_shape=jax.ShapeDtypeStruct((B, 8, KPAD), jnp.float32),
        compiler_params=pltpu.CompilerParams(
            dimension_semantics=("arbitrary",)),
    )(relT, anchT, scT)
    return _postprocess(o)


def kernel(rel_codes, scores, anchors):
    relT = jnp.pad(jnp.transpose(rel_codes, (0, 2, 1)),
                   ((0, 0), (0, 4), (0, NPAD - N)))
    anchT = jnp.pad(jnp.transpose(anchors, (1, 0)),
                    ((0, 4), (0, NPAD - N)))
    scT = jnp.pad(jnp.transpose(scores, (0, 2, 1)),
                  ((0, 0), (0, CPAD - C), (0, NPAD - N)),
                  constant_values=NEG)

    probs, coords, cnts = pl.pallas_call(
        _prep_body,
        grid=(B,),
        in_specs=[
            pl.BlockSpec((1, 8, NPAD), lambda b: (b, 0, 0)),
            pl.BlockSpec((8, NPAD), lambda b: (0, 0)),
            pl.BlockSpec((1, CPAD, NPAD), lambda b: (b, 0, 0)),
        ],
        out_specs=[
            pl.BlockSpec((1, CPAD, NPAD), lambda b: (b, 0, 0)),
            pl.BlockSpec((1, 8, NPAD), lambda b: (b, 0, 0)),
            pl.BlockSpec((1, CPAD, 16), lambda b: (b, 0, 0)),
        ],
        out_shape=[
            jax.ShapeDtypeStruct((B, CPAD, NPAD), jnp.float32),
            jax.ShapeDtypeStruct((B, 8, NPAD), jnp.float32),
            jax.ShapeDtypeStruct((B, CPAD, 16), jnp.float32),
        ],
        compiler_params=pltpu.CompilerParams(
            dimension_semantics=("arbitrary",)),
    )(relT, anchT, scT)

    sc_compact = _make_sc_compact()
    cprob, cy0, cx0, cy1, cx1 = sc_compact(
        probs.reshape(R, NPAD), coords, cnts.reshape(R, 16),
        jnp.asarray(LADDER, jnp.float32))

    o = pl.pallas_call(
        _rnms_body,
        out_shape=jax.ShapeDtypeStruct((B, 8, KPAD), jnp.float32),
    )(cprob, cy0, cx0, cy1, cx1, cnts)

    ok = jnp.min(o[:, 7, 0]) > 0.5
    fast = _postprocess(o)
    return jax.lax.cond(ok, lambda: fast,
                        lambda: _full_path(relT, anchT, scT))
